# baseline (speedup 1.0000x reference)
import jax
import jax.numpy as jnp
from jax import lax
from jax.experimental import pallas as pl
from jax.experimental.pallas import tpu as pltpu
from jax.experimental.pallas import tpu_sc as plsc

N_NODES = 10000
D_FEAT = 128
N_EDGES = 320000

NC = 2
NS = 16
DH = D_FEAT // NC
CHUNK = 256
EPT = 20480
E_PAD = EPT * NS
N_PAD = 10112
ROWS_PT = N_PAD // NS
N_CHUNKS = EPT // CHUNK
PHASES = 4
CPP = N_CHUNKS // PHASES
EPP = CPP * CHUNK


def _sc_kernel(xf0_hbm, xf1_hbm, src_hbm, dst_hbm, zeros_hbm, out_hbm,
               src_v, dst_v, buf0, buf1, x_sp, acc,
               zsem, xsem, gs0, gs1, ss0, ss1):
    cid = lax.axis_index("c")
    sid = lax.axis_index("s")
    rbase = sid * ROWS_PT

    @pl.when(cid == 0)
    def _():
        pltpu.async_copy(xf0_hbm.at[pl.ds(rbase, ROWS_PT)],
                         x_sp.at[pl.ds(rbase, ROWS_PT)], xsem)

    @pl.when(cid == 1)
    def _():
        pltpu.async_copy(xf1_hbm.at[pl.ds(rbase, ROWS_PT)],
                         x_sp.at[pl.ds(rbase, ROWS_PT)], xsem)

    xcopy = pltpu.make_async_copy(xf0_hbm.at[pl.ds(rbase, ROWS_PT)],
                                  x_sp.at[pl.ds(rbase, ROWS_PT)], xsem)
    zcopy = pltpu.async_copy(zeros_hbm.at[pl.ds(rbase, ROWS_PT)],
                             acc.at[pl.ds(rbase, ROWS_PT)], zsem)

    def stage(h):
        ebase = sid * EPT + h * EPP
        pltpu.sync_copy(src_hbm.at[pl.ds(ebase, EPP)], src_v)
        pltpu.sync_copy(dst_hbm.at[pl.ds(sid * N_CHUNKS + h * CPP, CPP)],
                        dst_v)

    def fire_g(j, buf, sem):
        hbm = (j % 3) == 2

        @pl.when(jnp.logical_and(hbm, cid == 0))
        def _():
            pltpu.async_copy(xf0_hbm.at[src_v.at[pl.ds(j * CHUNK, CHUNK)]],
                             buf, sem)

        @pl.when(jnp.logical_and(hbm, cid == 1))
        def _():
            pltpu.async_copy(xf1_hbm.at[src_v.at[pl.ds(j * CHUNK, CHUNK)]],
                             buf, sem)

        @pl.when(jnp.logical_not(hbm))
        def _():
            pltpu.async_copy(x_sp.at[src_v.at[pl.ds(j * CHUNK, CHUNK)]],
                             buf, sem)

    def drain(buf, sem):
        pltpu.make_async_copy(x_sp.at[pl.ds(0, CHUNK)], buf, sem).wait()

    def fire_s(j, buf, sem):
        pltpu.async_copy(buf, acc.at[dst_v.at[j]], sem, add=True)

    stage(0)
    xcopy.wait()
    zcopy.wait()
    plsc.subcore_barrier()

    def run_phase(last):
        fire_g(0, buf0, gs0)
        fire_g(1, buf1, gs1)

        def pipe(jp, _):
            j0 = 2 * jp
            drain(buf0, gs0)
            fire_s(j0, buf0, ss0)

            @pl.when(j0 + 2 < CPP)
            def _():
                drain(buf0, ss0)
                fire_g(j0 + 2, buf0, gs0)

            drain(buf1, gs1)
            fire_s(j0 + 1, buf1, ss1)

            @pl.when(j0 + 3 < CPP)
            def _():
                drain(buf1, ss1)
                fire_g(j0 + 3, buf1, gs1)

            return 0

        lax.fori_loop(0, CPP // 2, pipe, 0)
        drain(buf0, ss0)
        drain(buf1, ss1)

    for h in range(PHASES):
        run_phase(h == PHASES - 1)
        if h + 1 < PHASES:
            stage(h + 1)

    plsc.subcore_barrier()

    pltpu.sync_copy(acc.at[pl.ds(rbase, ROWS_PT)],
                    out_hbm.at[pl.ds(rbase, ROWS_PT), pl.ds(cid * DH, DH)])


@jax.jit
def kernel(x, edge_index):
    src = edge_index[0].astype(jnp.int32)
    dst = edge_index[1].astype(jnp.int32)

    pad = E_PAD - N_EDGES
    src = jnp.concatenate([src, jnp.zeros((pad,), jnp.int32)])
    dst = jnp.concatenate([dst, jnp.full((pad,), N_NODES, jnp.int32)])
    dst = dst.reshape(E_PAD // CHUNK, CHUNK)

    xr = x.reshape(N_NODES, NC, DH)
    xf0 = jnp.zeros((N_PAD, DH), jnp.float32).at[:N_NODES].set(xr[:, 0])
    xf1 = jnp.zeros((N_PAD, DH), jnp.float32).at[:N_NODES].set(xr[:, 1])

    zeros = jnp.zeros((N_PAD, DH), jnp.float32)

    mesh = plsc.VectorSubcoreMesh(core_axis_name="c", subcore_axis_name="s")
    out = pl.kernel(
        _sc_kernel,
        mesh=mesh,
        compiler_params=pltpu.CompilerParams(use_tc_tiling_on_sc=False),
        out_type=jax.ShapeDtypeStruct((N_PAD, D_FEAT), jnp.float32),
        scratch_types=[
            pltpu.VMEM((EPP,), jnp.int32),
            pltpu.VMEM((CPP, CHUNK), jnp.int32),
            pltpu.VMEM((CHUNK, DH), jnp.float32),
            pltpu.VMEM((CHUNK, DH), jnp.float32),
            pltpu.VMEM_SHARED((N_PAD, DH), jnp.float32),
            pltpu.VMEM_SHARED((N_PAD, DH), jnp.float32),
            pltpu.SemaphoreType.DMA,
            pltpu.SemaphoreType.DMA,
            pltpu.SemaphoreType.DMA,
            pltpu.SemaphoreType.DMA,
            pltpu.SemaphoreType.DMA,
            pltpu.SemaphoreType.DMA,
        ],
    )(xf0, xf1, src, dst, zeros)

    return out[:N_NODES]


# speedup vs baseline: 1.2214x; 1.2214x over previous
import jax
import jax.numpy as jnp
from jax import lax
from jax.experimental import pallas as pl
from jax.experimental.pallas import tpu as pltpu
from jax.experimental.pallas import tpu_sc as plsc

N_NODES = 10000
D_FEAT = 128
N_EDGES = 320000

NC = 2
NS = 16
DH = D_FEAT // NC
CHUNK = 256
EPT = 20480
E_PAD = EPT * NS
N_PAD = 10112
ROWS_PT = N_PAD // NS
N_CHUNKS = EPT // CHUNK
PHASES = 4
CPP = N_CHUNKS // PHASES
EPP = CPP * CHUNK


def _sc_kernel(xcols_hbm, src_hbm, dst_hbm, zeros_hbm, out_hbm,
               src_v, dst_v, buf0, buf1, x_sp, acc,
               zsem, xsem, gs0, gs1, ss0, ss1):
    cid = lax.axis_index("c")
    sid = lax.axis_index("s")
    rbase = sid * ROWS_PT

    xcopy = pltpu.async_copy(xcols_hbm.at[cid, pl.ds(rbase, ROWS_PT)],
                             x_sp.at[pl.ds(rbase, ROWS_PT)], xsem)
    zcopy = pltpu.async_copy(zeros_hbm.at[pl.ds(rbase, ROWS_PT)],
                             acc.at[pl.ds(rbase, ROWS_PT)], zsem)

    def stage(h):
        ebase = sid * EPT + h * EPP
        pltpu.sync_copy(src_hbm.at[pl.ds(ebase, EPP)], src_v)
        pltpu.sync_copy(dst_hbm.at[pl.ds(sid * N_CHUNKS + h * CPP, CPP)],
                        dst_v)

    def fire_g(j, buf, sem):
        pltpu.async_copy(x_sp.at[src_v.at[pl.ds(j * CHUNK, CHUNK)]],
                         buf, sem)

    def drain(buf, sem):
        pltpu.make_async_copy(x_sp.at[pl.ds(0, CHUNK)], buf, sem).wait()

    def fire_s(j, buf, sem):
        pltpu.async_copy(buf, acc.at[dst_v.at[j]], sem, add=True)

    stage(0)
    xcopy.wait()
    zcopy.wait()
    plsc.subcore_barrier()

    def run_phase(last):
        fire_g(0, buf0, gs0)
        fire_g(1, buf1, gs1)

        def pipe(jp, _):
            j0 = 2 * jp
            drain(buf0, gs0)
            fire_s(j0, buf0, ss0)

            @pl.when(j0 + 2 < CPP)
            def _():
                drain(buf0, ss0)
                fire_g(j0 + 2, buf0, gs0)

            drain(buf1, gs1)
            fire_s(j0 + 1, buf1, ss1)

            @pl.when(j0 + 3 < CPP)
            def _():
                drain(buf1, ss1)
                fire_g(j0 + 3, buf1, gs1)

            return 0

        lax.fori_loop(0, CPP // 2, pipe, 0)
        drain(buf0, ss0)
        drain(buf1, ss1)

    for h in range(PHASES):
        run_phase(h == PHASES - 1)
        if h + 1 < PHASES:
            stage(h + 1)

    plsc.subcore_barrier()

    pltpu.sync_copy(acc.at[pl.ds(rbase, ROWS_PT)],
                    out_hbm.at[pl.ds(rbase, ROWS_PT), pl.ds(cid * DH, DH)])


@jax.jit
def kernel(x, edge_index):
    src = edge_index[0].astype(jnp.int32)
    dst = edge_index[1].astype(jnp.int32)

    pad = E_PAD - N_EDGES
    src = jnp.concatenate([src, jnp.zeros((pad,), jnp.int32)])
    dst = jnp.concatenate([dst, jnp.full((pad,), N_NODES, jnp.int32)])
    dst = dst.reshape(E_PAD // CHUNK, CHUNK)

    xr = x.reshape(N_NODES, NC, DH).transpose(1, 0, 2)
    xcols = jnp.zeros((NC, N_PAD, DH), jnp.float32).at[:, :N_NODES].set(xr)

    zeros = jnp.zeros((N_PAD, DH), jnp.float32)

    mesh = plsc.VectorSubcoreMesh(core_axis_name="c", subcore_axis_name="s")
    out = pl.kernel(
        _sc_kernel,
        mesh=mesh,
        compiler_params=pltpu.CompilerParams(use_tc_tiling_on_sc=False),
        out_type=jax.ShapeDtypeStruct((N_PAD, D_FEAT), jnp.float32),
        scratch_types=[
            pltpu.VMEM((EPP,), jnp.int32),
            pltpu.VMEM((CPP, CHUNK), jnp.int32),
            pltpu.VMEM((CHUNK, DH), jnp.float32),
            pltpu.VMEM((CHUNK, DH), jnp.float32),
            pltpu.VMEM_SHARED((N_PAD, DH), jnp.float32),
            pltpu.VMEM_SHARED((N_PAD, DH), jnp.float32),
            pltpu.SemaphoreType.DMA,
            pltpu.SemaphoreType.DMA,
            pltpu.SemaphoreType.DMA,
            pltpu.SemaphoreType.DMA,
            pltpu.SemaphoreType.DMA,
            pltpu.SemaphoreType.DMA,
        ],
    )(xcols, src, dst, zeros)

    return out[:N_NODES]
